# baseline (speedup 1.0000x reference)
import functools

import jax
import jax.numpy as jnp
from jax import lax
from jax.experimental import pallas as pl
from jax.experimental.pallas import tpu as pltpu
from jax.experimental.pallas import tpu_sc as plsc

NC = 2
NS = 16
CHUNK = 128
G = 256
BN = 512
H = 128


def _round_up(v, m):
    return ((v + m - 1) // m) * m


def _sc_mesh():
    return plsc.VectorSubcoreMesh(core_axis_name="c", subcore_axis_name="s")


def _make_deg_kernel(n_pad, e_pad):
    rpt = n_pad // NS
    epw = e_pad // (NC * NS)
    nch = epw // CHUNK

    def body(dst_hbm, ones_hbm, out_hbm, acc_sh, ones_v, idx_v):
        c = lax.axis_index("c")
        s = lax.axis_index("s")
        rb = s * rpt
        pltpu.sync_copy(ones_hbm.at[pl.ds(rb, rpt)], acc_sh.at[pl.ds(rb, rpt)])
        pltpu.sync_copy(ones_hbm.at[pl.ds(0, CHUNK)], ones_v)
        plsc.subcore_barrier()
        eb = (c * NS + s) * epw

        def step(i, carry):
            off = eb + i * CHUNK
            pltpu.sync_copy(dst_hbm.at[pl.ds(off, CHUNK)], idx_v)
            pltpu.sync_copy(ones_v, acc_sh.at[idx_v], add=True)
            return carry

        lax.fori_loop(0, nch, step, 0)
        plsc.subcore_barrier()
        pltpu.sync_copy(acc_sh.at[pl.ds(rb, rpt)],
                        out_hbm.at[c, pl.ds(rb, rpt)])

    return pl.kernel(
        body,
        out_type=jax.ShapeDtypeStruct((NC, n_pad, 8), jnp.float32),
        mesh=_sc_mesh(),
        scratch_types=[
            pltpu.VMEM_SHARED((n_pad, 8), jnp.float32),
            pltpu.VMEM((CHUNK, 8), jnp.float32),
            pltpu.VMEM((CHUNK,), jnp.int32),
        ],
    )


def _make_agg_kernel(n_pad, e_pad):
    rpt = n_pad // NS
    epw = e_pad // (NC * NS)
    nch = epw // CHUNK

    def body(src_hbm, dst_hbm, g_hbm, out_hbm, acc_sh,
             si_a, di_a, si_b, di_b, r_a, r_b, sem_g, sem_s):
        c = lax.axis_index("c")
        s = lax.axis_index("s")
        rb = s * rpt
        pltpu.sync_copy(g_hbm.at[pl.ds(rb, rpt)], acc_sh.at[pl.ds(rb, rpt)])
        plsc.subcore_barrier()
        eb = (c * NS + s) * epw

        def load_gather(off, si, di, r):
            pltpu.sync_copy(src_hbm.at[pl.ds(off, CHUNK)], si)
            pltpu.sync_copy(dst_hbm.at[pl.ds(off, CHUNK)], di)
            pltpu.async_copy(g_hbm.at[si], r, sem_g).wait()

        load_gather(eb, si_a, di_a, r_a)

        def step(j, carry):
            off = eb + 2 * j * CHUNK
            d_a = pltpu.async_copy(r_a, acc_sh.at[di_a], sem_s, add=True)
            load_gather(off + CHUNK, si_b, di_b, r_b)
            d_a.wait()
            d_b = pltpu.async_copy(r_b, acc_sh.at[di_b], sem_s, add=True)
            load_gather(off + 2 * CHUNK, si_a, di_a, r_a)
            d_b.wait()
            return carry

        lax.fori_loop(0, nch // 2, step, 0)
        plsc.subcore_barrier()
        pltpu.sync_copy(acc_sh.at[pl.ds(rb, rpt)],
                        out_hbm.at[c, pl.ds(rb, rpt)])

    return pl.kernel(
        body,
        out_type=jax.ShapeDtypeStruct((NC, n_pad, H), jnp.float32),
        mesh=_sc_mesh(),
        scratch_types=[
            pltpu.VMEM_SHARED((n_pad, H), jnp.float32),
            pltpu.VMEM((CHUNK,), jnp.int32),
            pltpu.VMEM((CHUNK,), jnp.int32),
            pltpu.VMEM((CHUNK,), jnp.int32),
            pltpu.VMEM((CHUNK,), jnp.int32),
            pltpu.VMEM((CHUNK, H), jnp.float32),
            pltpu.VMEM((CHUNK, H), jnp.float32),
            pltpu.SemaphoreType.DMA,
            pltpu.SemaphoreType.DMA,
        ],
    )


def _dinv_masked(deg_ref, i, nreal):
    deg = deg_ref[0][:, 0:1] + deg_ref[1][:, 0:1] - 1.0
    rows = i * BN + lax.broadcasted_iota(jnp.int32, (BN, 1), 0)
    return jnp.where(rows < nreal, lax.rsqrt(deg), 0.0)


def _t0_body(deg_ref, x_ref, w_ref, o_ref, *, nreal):
    dinv = _dinv_masked(deg_ref, pl.program_id(0), nreal)
    o_ref[...] = dinv * jnp.dot(x_ref[...], w_ref[...],
                                preferred_element_type=jnp.float32)


def _tmid_body(deg_ref, a_ref, g_ref, b_ref, w_ref, o_ref, *, nreal):
    dinv = _dinv_masked(deg_ref, pl.program_id(0), nreal)
    xb = jnp.tanh(dinv * (a_ref[0] + a_ref[1] - g_ref[...]) + b_ref[...])
    o_ref[...] = dinv * jnp.dot(xb, w_ref[...],
                                preferred_element_type=jnp.float32)


def _head_body(deg_ref, a_ref, g_ref, b2_ref, w1_ref, b1_ref, w2_ref, b2f_ref,
               w3_ref, b3_ref, batch_ref, ia_ref, ib_ref,
               sums_ref, cnts_ref, xutil_ref, pairs_ref, *, nreal, nb, npairs):
    i = pl.program_id(0)
    dinv = _dinv_masked(deg_ref, i, nreal)
    xb = jnp.tanh(dinv * (a_ref[0] + a_ref[1] - g_ref[...]) + b2_ref[...])
    h1 = jnp.tanh(jnp.dot(xb, w1_ref[...],
                          preferred_element_type=jnp.float32) + b1_ref[...])
    h2 = jnp.tanh(jnp.dot(h1, w2_ref[...],
                          preferred_element_type=jnp.float32) + b2f_ref[...])
    h3 = jnp.dot(h2, w3_ref[...],
                 preferred_element_type=jnp.float32) + b3_ref[...]

    rows = i * BN + lax.broadcasted_iota(jnp.int32, (BN, 1), 0)
    oh = ((batch_ref[...] == lax.broadcasted_iota(jnp.int32, (BN, G), 1))
          & (rows < nreal)).astype(jnp.float32)
    s_c = lax.dot_general(oh, h3, (((0,), (0,)), ((), ())),
                          preferred_element_type=jnp.float32)
    c_c = jnp.sum(oh, axis=0)[:, None]

    @pl.when(i == 0)
    def _init():
        sums_ref[...] = jnp.zeros_like(sums_ref)
        cnts_ref[...] = jnp.zeros_like(cnts_ref)

    sums_ref[...] += s_c
    cnts_ref[...] += c_c

    @pl.when(i == nb - 1)
    def _finish():
        util = sums_ref[...] / jnp.maximum(cnts_ref[...], 1.0)
        xutil_ref[...] = util
        gi = lax.broadcasted_iota(jnp.int32, (npairs, G), 1)
        oha = (ia_ref[...] == gi).astype(jnp.float32)
        ohb = (ib_ref[...] == gi).astype(jnp.float32)
        diff = jnp.dot(ohb - oha, util, preferred_element_type=jnp.float32)
        pairs_ref[...] = jax.nn.sigmoid(diff)


def kernel(x, edge_index, batch, idx_a, idx_b, params):
    n, d_in = x.shape
    e = edge_index.shape[1]
    p = idx_a.shape[0]
    n_pad = _round_up(n + 1, BN)
    e_pad = _round_up(e, NC * NS * CHUNK * 2)
    nb = n_pad // BN

    src = edge_index[0].astype(jnp.int32)
    dst = edge_index[1].astype(jnp.int32)
    pad_e = e_pad + CHUNK - e
    src_p = jnp.concatenate([src, jnp.full((pad_e,), n, jnp.int32)])
    dst_p = jnp.concatenate([dst, jnp.full((pad_e,), n, jnp.int32)])
    x_p = jnp.zeros((n_pad, d_in), jnp.float32).at[:n].set(x)
    batch_p = jnp.full((n_pad, 1), G, jnp.int32).at[:n, 0].set(
        batch.astype(jnp.int32))
    ones8 = jnp.ones((n_pad, 8), jnp.float32)
    ia = idx_a.astype(jnp.int32)[:, None]
    ib = idx_b.astype(jnp.int32)[:, None]

    (w0, b0) = params['conv_in']
    convs = params['convs']
    (wf1, bf1) = params['fc1']
    (wf2, bf2) = params['fc2']
    (wf3, bf3) = params['fc3']

    deg = _make_deg_kernel(n_pad, e_pad)(dst_p, ones8)
    agg_call = _make_agg_kernel(n_pad, e_pad)

    spec_deg = pl.BlockSpec((2, BN, 8), lambda i: (0, i, 0))
    spec_row = pl.BlockSpec((BN, H), lambda i: (i, 0))
    spec_a = pl.BlockSpec((2, BN, H), lambda i: (0, i, 0))
    spec_w = pl.BlockSpec((H, H), lambda i: (0, 0))
    spec_b = pl.BlockSpec((1, H), lambda i: (0, 0))
    row_out = jax.ShapeDtypeStruct((n_pad, H), jnp.float32)

    g = pl.pallas_call(
        functools.partial(_t0_body, nreal=n),
        grid=(nb,),
        in_specs=[spec_deg, spec_row, spec_w],
        out_specs=spec_row,
        out_shape=row_out,
    )(deg, x_p, w0)

    biases = [b0] + [b for (_, b) in convs]
    for li, (w, _) in enumerate(convs):
        a = agg_call(src_p, dst_p, g)
        g = pl.pallas_call(
            functools.partial(_tmid_body, nreal=n),
            grid=(nb,),
            in_specs=[spec_deg, spec_a, spec_row, spec_b, spec_w],
            out_specs=spec_row,
            out_shape=row_out,
        )(deg, a, g, biases[li][None, :], w)

    a = agg_call(src_p, dst_p, g)

    const = lambda shape: pl.BlockSpec(shape, lambda i: tuple(0 for _ in shape))
    sums, cnts, xutil, pairs = pl.pallas_call(
        functools.partial(_head_body, nreal=n, nb=nb, npairs=p),
        grid=(nb,),
        in_specs=[spec_deg, spec_a, spec_row, spec_b,
                  spec_w, spec_b,
                  const((H, 32)), const((1, 32)),
                  const((32, 1)), const((1, 1)),
                  pl.BlockSpec((BN, 1), lambda i: (i, 0)),
                  const((p, 1)), const((p, 1))],
        out_specs=[const((G, 1)), const((G, 1)), const((G, 1)), const((p, 1))],
        out_shape=[jax.ShapeDtypeStruct((G, 1), jnp.float32),
                   jax.ShapeDtypeStruct((G, 1), jnp.float32),
                   jax.ShapeDtypeStruct((G, 1), jnp.float32),
                   jax.ShapeDtypeStruct((p, 1), jnp.float32)],
    )(deg, a, g, biases[-1][None, :], wf1, bf1[None, :], wf2, bf2[None, :],
      wf3, bf3[None, :], batch_p, ia, ib)

    return (pairs[:, 0], xutil)


# speedup vs baseline: 1.7889x; 1.7889x over previous
import functools

import jax
import jax.numpy as jnp
from jax import lax
from jax.experimental import pallas as pl
from jax.experimental.pallas import tpu as pltpu
from jax.experimental.pallas import tpu_sc as plsc

NC = 2
NS = 16
CHUNK = 128
G = 256
BN = 512
H = 128
HALF = H // NC


def _round_up(v, m):
    return ((v + m - 1) // m) * m


def _sc_mesh():
    return plsc.VectorSubcoreMesh(core_axis_name="c", subcore_axis_name="s")


def _make_deg_kernel(n_pad, e_pad):
    rpt = n_pad // NS
    epw = e_pad // (NC * NS)
    nch = epw // CHUNK

    def body(dst_hbm, ones_hbm, out_hbm, acc_sh, ones_v, idx_v):
        c = lax.axis_index("c")
        s = lax.axis_index("s")
        rb = s * rpt
        pltpu.sync_copy(ones_hbm.at[pl.ds(rb, rpt)], acc_sh.at[pl.ds(rb, rpt)])
        pltpu.sync_copy(ones_hbm.at[pl.ds(0, CHUNK)], ones_v)
        plsc.subcore_barrier()
        eb = (c * NS + s) * epw

        def step(i, carry):
            off = eb + i * CHUNK
            pltpu.sync_copy(dst_hbm.at[pl.ds(off, CHUNK)], idx_v)
            pltpu.sync_copy(ones_v, acc_sh.at[idx_v], add=True)
            return carry

        lax.fori_loop(0, nch, step, 0)
        plsc.subcore_barrier()
        pltpu.sync_copy(acc_sh.at[pl.ds(rb, rpt)],
                        out_hbm.at[c, pl.ds(rb, rpt)])

    return pl.kernel(
        body,
        out_type=jax.ShapeDtypeStruct((NC, n_pad, 8), jnp.float32),
        mesh=_sc_mesh(),
        scratch_types=[
            pltpu.VMEM_SHARED((n_pad, 8), jnp.float32),
            pltpu.VMEM((CHUNK, 8), jnp.float32),
            pltpu.VMEM((CHUNK,), jnp.int32),
        ],
    )


def _make_agg_kernel(n_pad, e_pad):
    rpt = n_pad // NS
    epw = e_pad // NS
    nch = epw // CHUNK

    def body(src_hbm, dst_hbm, g_hbm, out_hbm, g_sh, acc_sh,
             si_a, di_a, si_b, di_b, r_a, r_b, sem_g, sem_s):
        c = lax.axis_index("c")
        s = lax.axis_index("s")
        rb = s * rpt
        pltpu.sync_copy(g_hbm.at[c, pl.ds(rb, rpt)], g_sh.at[pl.ds(rb, rpt)])
        pltpu.sync_copy(g_hbm.at[c, pl.ds(rb, rpt)], acc_sh.at[pl.ds(rb, rpt)])
        plsc.subcore_barrier()
        eb = s * epw

        def load_gather(off, si, di, r):
            pltpu.sync_copy(src_hbm.at[pl.ds(off, CHUNK)], si)
            pltpu.sync_copy(dst_hbm.at[pl.ds(off, CHUNK)], di)
            pltpu.async_copy(g_sh.at[si], r, sem_g).wait()

        load_gather(eb, si_a, di_a, r_a)

        def step(j, carry):
            off = eb + 2 * j * CHUNK
            d_a = pltpu.async_copy(r_a, acc_sh.at[di_a], sem_s, add=True)
            load_gather(off + CHUNK, si_b, di_b, r_b)
            d_a.wait()
            d_b = pltpu.async_copy(r_b, acc_sh.at[di_b], sem_s, add=True)
            load_gather(off + 2 * CHUNK, si_a, di_a, r_a)
            d_b.wait()
            return carry

        lax.fori_loop(0, nch // 2, step, 0)
        plsc.subcore_barrier()
        pltpu.sync_copy(acc_sh.at[pl.ds(rb, rpt)],
                        out_hbm.at[c, pl.ds(rb, rpt)])

    return pl.kernel(
        body,
        out_type=jax.ShapeDtypeStruct((NC, n_pad, HALF), jnp.float32),
        mesh=_sc_mesh(),
        scratch_types=[
            pltpu.VMEM_SHARED((n_pad, HALF), jnp.float32),
            pltpu.VMEM_SHARED((n_pad, HALF), jnp.float32),
            pltpu.VMEM((CHUNK,), jnp.int32),
            pltpu.VMEM((CHUNK,), jnp.int32),
            pltpu.VMEM((CHUNK,), jnp.int32),
            pltpu.VMEM((CHUNK,), jnp.int32),
            pltpu.VMEM((CHUNK, HALF), jnp.float32),
            pltpu.VMEM((CHUNK, HALF), jnp.float32),
            pltpu.SemaphoreType.DMA,
            pltpu.SemaphoreType.DMA,
        ],
    )


def _dinv_masked(deg_ref, i, nreal):
    deg = deg_ref[0][:, 0:1] + deg_ref[1][:, 0:1] - 1.0
    rows = i * BN + lax.broadcasted_iota(jnp.int32, (BN, 1), 0)
    return jnp.where(rows < nreal, lax.rsqrt(deg), 0.0)


def _split(o_ref, res):
    o_ref[0] = res[:, :HALF]
    o_ref[1] = res[:, HALF:]


def _t0_body(deg_ref, x_ref, w_ref, o_ref, *, nreal):
    dinv = _dinv_masked(deg_ref, pl.program_id(0), nreal)
    _split(o_ref, dinv * jnp.dot(x_ref[...], w_ref[...],
                                 preferred_element_type=jnp.float32))


def _tmid_body(deg_ref, a_ref, b_ref, w_ref, o_ref, *, nreal):
    dinv = _dinv_masked(deg_ref, pl.program_id(0), nreal)
    agg = jnp.concatenate([a_ref[0], a_ref[1]], axis=1)
    xb = jnp.tanh(dinv * agg + b_ref[...])
    _split(o_ref, dinv * jnp.dot(xb, w_ref[...],
                                 preferred_element_type=jnp.float32))


def _head_body(deg_ref, a_ref, b2_ref, w1_ref, b1_ref, w2_ref, b2f_ref,
               w3_ref, b3_ref, batch_ref, ia_ref, ib_ref,
               sums_ref, cnts_ref, xutil_ref, pairs_ref, *, nreal, nb, npairs):
    i = pl.program_id(0)
    dinv = _dinv_masked(deg_ref, i, nreal)
    agg = jnp.concatenate([a_ref[0], a_ref[1]], axis=1)
    xb = jnp.tanh(dinv * agg + b2_ref[...])
    h1 = jnp.tanh(jnp.dot(xb, w1_ref[...],
                          preferred_element_type=jnp.float32) + b1_ref[...])
    h2 = jnp.tanh(jnp.dot(h1, w2_ref[...],
                          preferred_element_type=jnp.float32) + b2f_ref[...])
    h3 = jnp.dot(h2, w3_ref[...],
                 preferred_element_type=jnp.float32) + b3_ref[...]

    rows = i * BN + lax.broadcasted_iota(jnp.int32, (BN, 1), 0)
    oh = ((batch_ref[...] == lax.broadcasted_iota(jnp.int32, (BN, G), 1))
          & (rows < nreal)).astype(jnp.float32)
    s_c = lax.dot_general(oh, h3, (((0,), (0,)), ((), ())),
                          preferred_element_type=jnp.float32)
    c_c = jnp.sum(oh, axis=0)[:, None]

    @pl.when(i == 0)
    def _init():
        sums_ref[...] = jnp.zeros_like(sums_ref)
        cnts_ref[...] = jnp.zeros_like(cnts_ref)

    sums_ref[...] += s_c
    cnts_ref[...] += c_c

    @pl.when(i == nb - 1)
    def _finish():
        util = sums_ref[...] / jnp.maximum(cnts_ref[...], 1.0)
        xutil_ref[...] = util
        gi = lax.broadcasted_iota(jnp.int32, (npairs, G), 1)
        oha = (ia_ref[...] == gi).astype(jnp.float32)
        ohb = (ib_ref[...] == gi).astype(jnp.float32)
        diff = jnp.dot(ohb - oha, util, preferred_element_type=jnp.float32)
        pairs_ref[...] = jax.nn.sigmoid(diff)


def kernel(x, edge_index, batch, idx_a, idx_b, params):
    n, d_in = x.shape
    e = edge_index.shape[1]
    p = idx_a.shape[0]
    n_pad = _round_up(n + 1, BN)
    e_pad = _round_up(e, NC * NS * CHUNK * 2)
    nb = n_pad // BN

    src = edge_index[0].astype(jnp.int32)
    dst = edge_index[1].astype(jnp.int32)
    pad_e = e_pad + CHUNK - e
    src_p = jnp.concatenate([src, jnp.full((pad_e,), n, jnp.int32)])
    dst_p = jnp.concatenate([dst, jnp.full((pad_e,), n, jnp.int32)])
    x_p = jnp.zeros((n_pad, d_in), jnp.float32).at[:n].set(x)
    batch_p = jnp.full((n_pad, 1), G, jnp.int32).at[:n, 0].set(
        batch.astype(jnp.int32))
    ones8 = jnp.ones((n_pad, 8), jnp.float32)
    ia = idx_a.astype(jnp.int32)[:, None]
    ib = idx_b.astype(jnp.int32)[:, None]

    (w0, b0) = params['conv_in']
    convs = params['convs']
    (wf1, bf1) = params['fc1']
    (wf2, bf2) = params['fc2']
    (wf3, bf3) = params['fc3']

    deg = _make_deg_kernel(n_pad, e_pad)(dst_p, ones8)
    agg_call = _make_agg_kernel(n_pad, e_pad)

    spec_deg = pl.BlockSpec((2, BN, 8), lambda i: (0, i, 0))
    spec_split = pl.BlockSpec((2, BN, HALF), lambda i: (0, i, 0))
    spec_row = pl.BlockSpec((BN, H), lambda i: (i, 0))
    spec_w = pl.BlockSpec((H, H), lambda i: (0, 0))
    spec_b = pl.BlockSpec((1, H), lambda i: (0, 0))
    split_out = jax.ShapeDtypeStruct((NC, n_pad, HALF), jnp.float32)

    g = pl.pallas_call(
        functools.partial(_t0_body, nreal=n),
        grid=(nb,),
        in_specs=[spec_deg, spec_row, spec_w],
        out_specs=spec_split,
        out_shape=split_out,
    )(deg, x_p, w0)

    biases = [b0] + [b for (_, b) in convs]
    for li, (w, _) in enumerate(convs):
        a = agg_call(src_p, dst_p, g)
        g = pl.pallas_call(
            functools.partial(_tmid_body, nreal=n),
            grid=(nb,),
            in_specs=[spec_deg, spec_split, spec_b, spec_w],
            out_specs=spec_split,
            out_shape=split_out,
        )(deg, a, biases[li][None, :], w)

    a = agg_call(src_p, dst_p, g)

    const = lambda shape: pl.BlockSpec(shape, lambda i: tuple(0 for _ in shape))
    sums, cnts, xutil, pairs = pl.pallas_call(
        functools.partial(_head_body, nreal=n, nb=nb, npairs=p),
        grid=(nb,),
        in_specs=[spec_deg, spec_split, spec_b,
                  spec_w, spec_b,
                  const((H, 32)), const((1, 32)),
                  const((32, 1)), const((1, 1)),
                  pl.BlockSpec((BN, 1), lambda i: (i, 0)),
                  const((p, 1)), const((p, 1))],
        out_specs=[const((G, 1)), const((G, 1)), const((G, 1)), const((p, 1))],
        out_shape=[jax.ShapeDtypeStruct((G, 1), jnp.float32),
                   jax.ShapeDtypeStruct((G, 1), jnp.float32),
                   jax.ShapeDtypeStruct((G, 1), jnp.float32),
                   jax.ShapeDtypeStruct((p, 1), jnp.float32)],
    )(deg, a, biases[-1][None, :], wf1, bf1[None, :], wf2, bf2[None, :],
      wf3, bf3[None, :], batch_p, ia, ib)

    return (pairs[:, 0], xutil)
